# baseline (speedup 1.0000x reference)
import jax
import jax.numpy as jnp
from jax.experimental import pallas as pl
from jax.experimental.pallas import tpu as pltpu

_THRESHOLD = 0.5
_NEGPOS_RATIO = 3
_VAR0, _VAR1 = 0.1, 0.2
_W = 512


def _chunks(P):
    return [(c0, min(_W, P - c0)) for c0 in range(0, P, _W)]


def _padw(x, cw):
    return x if cw == _W else jnp.pad(x, ((0, 0), (0, _W - cw)))


def _match_body(tgt_ref, loc_ref, pri_ref, ct_ref, stats_ref, bto_ref,
                bti_ref):
    P = pri_ref.shape[1]
    T = tgt_ref.shape[1]

    tg = tgt_ref[0]
    tx1, ty1 = tg[:, 0:1], tg[:, 1:2]
    tx2, ty2 = tg[:, 2:3], tg[:, 3:4]
    area_a = (tx2 - tx1) * (ty2 - ty1)
    coords = jnp.transpose(tg[:, 0:8])

    rmax = jnp.full((T, 1), -1.0, jnp.float32)
    bpi = jnp.zeros((T, 1), jnp.int32)
    for c0, cw in _chunks(P):
        pr = pri_ref[:, c0:c0 + cw]
        cx, cy, w, h = pr[0:1], pr[1:2], pr[2:3], pr[3:4]
        pfx1 = cx - w / 2.0
        pfy1 = cy - h / 2.0
        pfx2 = cx + w / 2.0
        pfy2 = cy + h / 2.0
        area_b = (pfx2 - pfx1) * (pfy2 - pfy1)
        iw = jnp.maximum(jnp.minimum(tx2, pfx2) - jnp.maximum(tx1, pfx1), 0.0)
        ih = jnp.maximum(jnp.minimum(ty2, pfy2) - jnp.maximum(ty1, pfy1), 0.0)
        inter = iw * ih
        ov = inter / (area_a + area_b - inter)
        cmax = jnp.max(ov, axis=1, keepdims=True)
        gi = jax.lax.broadcasted_iota(jnp.int32, (T, cw), 1) + c0
        cidx = jnp.min(jnp.where(ov == cmax, gi, jnp.int32(2 ** 30)),
                       axis=1, keepdims=True)
        take = cmax > rmax
        rmax = jnp.where(take, cmax, rmax)
        bpi = jnp.where(take, cidx, bpi)
        t_iota2 = jax.lax.broadcasted_iota(jnp.int32, (T, cw), 0)
        bto = jnp.max(ov, axis=0, keepdims=True)
        bti = jnp.min(jnp.where(ov == bto, t_iota2, T + 1),
                      axis=0, keepdims=True)
        bto_ref[0:1, c0:c0 + cw] = bto
        bti_ref[0:1, c0:c0 + cw] = bti

    lossl_acc = jnp.zeros((1, _W), jnp.float32)
    npos_acc = jnp.zeros((1, _W), jnp.float32)
    for c0, cw in _chunks(P):
        t_iota2 = jax.lax.broadcasted_iota(jnp.int32, (T, cw), 0)
        bto = bto_ref[0:1, c0:c0 + cw]
        bti = bti_ref[0:1, c0:c0 + cw]
        gi = jax.lax.broadcasted_iota(jnp.int32, (T, cw), 1) + c0
        hit = bpi == gi
        t_win = jnp.max(jnp.where(hit, t_iota2, -1), axis=0, keepdims=True)
        forced = t_win >= 0
        bti = jnp.where(forced, t_win, bti)
        bto = jnp.where(forced, 2.0, bto)

        sel = (bti == t_iota2).astype(jnp.float32)
        picked = jax.lax.dot_general(
            coords, sel, (((1,), (0,)), ((), ())),
            preferred_element_type=jnp.float32)
        mx1, my1 = picked[0:1], picked[1:2]
        mx2, my2 = picked[2:3], picked[3:4]
        mlab = picked[4:5]

        pos = bto >= _THRESHOLD
        npos_acc += _padw(pos.astype(jnp.float32), cw)
        cti = jnp.where(pos, mlab + 1.0, 0.0).astype(jnp.int32)
        cwp = ((cw + 127) // 128) * 128
        ct_ref[0, 0:1, c0:c0 + cwp] = (
            jnp.pad(cti, ((0, 0), (0, cwp - cw))) if cwp != cw else cti)

        pr = pri_ref[:, c0:c0 + cw]
        cx, cy, w, h = pr[0:1], pr[1:2], pr[2:3], pr[3:4]
        l0 = ((mx1 + mx2) / 2.0 - cx) / (_VAR0 * w)
        l1 = ((my1 + my2) / 2.0 - cy) / (_VAR0 * h)
        l2 = jnp.log((mx2 - mx1) / w) / _VAR1
        l3 = jnp.log((my2 - my1) / h) / _VAR1

        for j, lj in enumerate((l0, l1, l2, l3)):
            d = loc_ref[0][j:j + 1, c0:c0 + cw] - lj
            ad = jnp.abs(d)
            sl1 = jnp.where(ad < 1.0, 0.5 * d * d, ad - 0.5)
            lossl_acc += _padw(jnp.where(pos, sl1, 0.0), cw)

    lossl_b = jnp.sum(lossl_acc)
    npos_b = jnp.sum(npos_acc)
    lane = jax.lax.broadcasted_iota(jnp.int32, (1, 128), 1)
    stats_ref[0, 0:1, :] = jnp.where(
        lane == 0, lossl_b, jnp.where(lane == 2, npos_b, 0.0))


def _ce_mine_body(conf_ref, ct_ref, stats_ref, out_ref, lcb_scr, acc_scr):
    b = pl.program_id(0)
    nb = pl.num_programs(0)
    C = conf_ref.shape[1]
    P = conf_ref.shape[2]
    lane = jax.lax.broadcasted_iota(jnp.int32, (1, 128), 1)

    lcp_acc = jnp.zeros((1, _W), jnp.float32)
    vals = []
    for c0, cw in _chunks(P):
        c = conf_ref[0][:, c0:c0 + cw]
        cti = ct_ref[0][0:1, c0:c0 + cw]
        pos = cti > 0
        m = jnp.max(c, axis=0, keepdims=True)
        e = jnp.exp(c - m)
        s = jax.lax.dot_general(
            jnp.ones((1, C), jnp.float32), e, (((1,), (0,)), ((), ())),
            preferred_element_type=jnp.float32)
        lse = m + jnp.log(s)
        cls_iota = jax.lax.broadcasted_iota(jnp.int32, (C, cw), 0)
        ltgt = jnp.sum(jnp.where(cls_iota == cti, c, 0.0),
                       axis=0, keepdims=True)
        ce = lse - ltgt
        lcp_acc += _padw(jnp.where(pos, ce, 0.0), cw)
        cwp = ((cw + 127) // 128) * 128
        val = jnp.maximum(ce, 0.0)
        vals.append(jnp.pad(val, ((0, 0), (0, cwp - cw)))
                    if cwp != cw else val)

    lcb_scr[pl.ds(b, 1), :] = jnp.concatenate(vals, axis=1)

    @pl.when(b == 0)
    def _():
        acc_scr[0:1, :] = jnp.zeros((1, 128), jnp.float32)

    acc_scr[0:1, :] += jnp.where(lane == 1, jnp.sum(lcp_acc), 0.0)

    @pl.when(b == nb - 1)
    def _():
        stats = stats_ref[:, :]
        tot = jnp.sum(stats, axis=0, keepdims=True) + acc_scr[0:1, :]
        ll = jnp.sum(jnp.where(lane == 0, tot, 0.0))
        lcp = jnp.sum(jnp.where(lane == 1, tot, 0.0))
        n_tot = jnp.sum(jnp.where(lane == 2, tot, 0.0))

        kf = jnp.minimum(jnp.float32(_NEGPOS_RATIO) * stats[:, 2:3],
                         jnp.float32(P - 1))
        v = lcb_scr[:, :]

        lo0 = jnp.zeros((nb, 1), jnp.int32)
        hi0 = jnp.full((nb, 1), 0x7F800000, jnp.int32)

        def step(_, carry):
            lo, hi = carry
            mid = lo + ((hi - lo + 1) >> 1)
            t = jax.lax.bitcast_convert_type(mid, jnp.float32)
            cnt = jnp.sum(jnp.where(v >= t, 1.0, 0.0), axis=1, keepdims=True)
            ok = cnt >= kf
            return jnp.where(ok, mid, lo), jnp.where(ok, hi, mid - 1)

        lo, _hi = jax.lax.fori_loop(0, 31, step, (lo0, hi0))
        tk = jax.lax.bitcast_convert_type(lo, jnp.float32)
        gt = v > tk
        sum_gt = jnp.sum(jnp.where(gt, v, 0.0), axis=1, keepdims=True)
        cnt_gt = jnp.sum(jnp.where(gt, 1.0, 0.0), axis=1, keepdims=True)
        lcn = jnp.sum(sum_gt + (kf - cnt_gt) * tk)

        out_ref[0:1, :] = jnp.where(
            lane == 0, ll / n_tot,
            jnp.where(lane == 1, (lcp + lcn) / n_tot, 0.0))


def _mbox_loss(loc_data, conf_data, priors, targets, interpret=False):
    bs, P, C = conf_data.shape
    T = targets.shape[1]
    Ppad = ((P + 127) // 128) * 128
    loc_t = jnp.swapaxes(loc_data, 1, 2)
    conf_t = jnp.swapaxes(conf_data, 1, 2)
    pri_t = priors[:P, :].T
    tgt_p = jnp.pad(targets, ((0, 0), (0, 0), (0, 128 - targets.shape[2])))

    ct, stats = pl.pallas_call(
        _match_body,
        grid=(bs,),
        in_specs=[
            pl.BlockSpec((1, T, 128), lambda b: (b, 0, 0)),
            pl.BlockSpec((1, 4, P), lambda b: (b, 0, 0)),
            pl.BlockSpec((4, P), lambda b: (0, 0)),
        ],
        out_specs=[
            pl.BlockSpec((1, 1, Ppad), lambda b: (b, 0, 0)),
            pl.BlockSpec((1, 1, 128), lambda b: (b, 0, 0)),
        ],
        out_shape=[
            jax.ShapeDtypeStruct((bs, 1, Ppad), jnp.int32),
            jax.ShapeDtypeStruct((bs, 1, 128), jnp.float32),
        ],
        scratch_shapes=[pltpu.VMEM((1, P), jnp.float32),
                        pltpu.VMEM((1, P), jnp.int32)],
        compiler_params=pltpu.CompilerParams(
            dimension_semantics=("arbitrary",)),
        interpret=interpret,
    )(tgt_p, loc_t, pri_t)

    out = pl.pallas_call(
        _ce_mine_body,
        grid=(bs,),
        in_specs=[
            pl.BlockSpec((1, C, P), lambda b: (b, 0, 0)),
            pl.BlockSpec((1, 1, Ppad), lambda b: (b, 0, 0)),
            pl.BlockSpec((bs, 128), lambda b: (0, 0)),
        ],
        out_specs=pl.BlockSpec((1, 128), lambda b: (0, 0)),
        out_shape=jax.ShapeDtypeStruct((1, 128), jnp.float32),
        scratch_shapes=[
            pltpu.VMEM((bs, Ppad), jnp.float32),
            pltpu.VMEM((1, 128), jnp.float32),
        ],
        compiler_params=pltpu.CompilerParams(
            dimension_semantics=("arbitrary",)),
        interpret=interpret,
    )(conf_t, ct, stats.reshape(bs, 128))
    return out[0, 0], out[0, 1]


def kernel(loc_data, conf_data, priors, targets):
    return _mbox_loss(loc_data, conf_data, priors, targets)


# speedup vs baseline: 1.0439x; 1.0439x over previous
import jax
import jax.numpy as jnp
from jax.experimental import pallas as pl
from jax.experimental.pallas import tpu as pltpu

_THRESHOLD = 0.5
_NEGPOS_RATIO = 3
_VAR0, _VAR1 = 0.1, 0.2
_W = 512


def _chunks(P):
    return [(c0, min(_W, P - c0)) for c0 in range(0, P, _W)]


def _padw(x, cw):
    return x if cw == _W else jnp.pad(x, ((0, 0), (0, _W - cw)))


def _match_body(tgt_ref, loc_ref, pri_ref, ct_ref, stats_ref, ov_ref):
    P = pri_ref.shape[1]
    T = tgt_ref.shape[1]

    tg = tgt_ref[0]
    tx1, ty1 = tg[:, 0:1], tg[:, 1:2]
    tx2, ty2 = tg[:, 2:3], tg[:, 3:4]
    area_a = (tx2 - tx1) * (ty2 - ty1)
    coords = jnp.transpose(tg[:, 0:8])

    rmax = jnp.full((T, 1), -1.0, jnp.float32)
    bpi = jnp.zeros((T, 1), jnp.int32)
    for c0, cw in _chunks(P):
        pr = pri_ref[:, c0:c0 + cw]
        cx, cy, w, h = pr[0:1], pr[1:2], pr[2:3], pr[3:4]
        pfx1 = cx - w / 2.0
        pfy1 = cy - h / 2.0
        pfx2 = cx + w / 2.0
        pfy2 = cy + h / 2.0
        area_b = (pfx2 - pfx1) * (pfy2 - pfy1)
        iw = jnp.maximum(jnp.minimum(tx2, pfx2) - jnp.maximum(tx1, pfx1), 0.0)
        ih = jnp.maximum(jnp.minimum(ty2, pfy2) - jnp.maximum(ty1, pfy1), 0.0)
        inter = iw * ih
        ov = inter / (area_a + area_b - inter)
        ov_ref[:, c0:c0 + cw] = ov
        cmax = jnp.max(ov, axis=1, keepdims=True)
        gi = jax.lax.broadcasted_iota(jnp.int32, (T, cw), 1) + c0
        cidx = jnp.min(jnp.where(ov == cmax, gi, jnp.int32(2 ** 30)),
                       axis=1, keepdims=True)
        take = cmax > rmax
        rmax = jnp.where(take, cmax, rmax)
        bpi = jnp.where(take, cidx, bpi)

    lossl_acc = jnp.zeros((1, _W), jnp.float32)
    npos_acc = jnp.zeros((1, _W), jnp.float32)
    for c0, cw in _chunks(P):
        ov = ov_ref[:, c0:c0 + cw]
        t_iota2 = jax.lax.broadcasted_iota(jnp.int32, (T, cw), 0)
        bto = jnp.max(ov, axis=0, keepdims=True)
        bti = jnp.min(jnp.where(ov == bto, t_iota2, T + 1),
                      axis=0, keepdims=True)
        gi = jax.lax.broadcasted_iota(jnp.int32, (T, cw), 1) + c0
        hit = bpi == gi
        t_win = jnp.max(jnp.where(hit, t_iota2, -1), axis=0, keepdims=True)
        forced = t_win >= 0
        bti = jnp.where(forced, t_win, bti)
        bto = jnp.where(forced, 2.0, bto)

        sel = (bti == t_iota2).astype(jnp.float32)
        picked = jax.lax.dot_general(
            coords, sel, (((1,), (0,)), ((), ())),
            preferred_element_type=jnp.float32)
        mx1, my1 = picked[0:1], picked[1:2]
        mx2, my2 = picked[2:3], picked[3:4]
        mlab = picked[4:5]

        pos = bto >= _THRESHOLD
        npos_acc += _padw(pos.astype(jnp.float32), cw)
        cti = jnp.where(pos, mlab + 1.0, 0.0).astype(jnp.int32)
        cwp = ((cw + 127) // 128) * 128
        ct_ref[0, 0:1, c0:c0 + cwp] = (
            jnp.pad(cti, ((0, 0), (0, cwp - cw))) if cwp != cw else cti)

        pr = pri_ref[:, c0:c0 + cw]
        cx, cy, w, h = pr[0:1], pr[1:2], pr[2:3], pr[3:4]
        l0 = ((mx1 + mx2) / 2.0 - cx) / (_VAR0 * w)
        l1 = ((my1 + my2) / 2.0 - cy) / (_VAR0 * h)
        l2 = jnp.log((mx2 - mx1) / w) / _VAR1
        l3 = jnp.log((my2 - my1) / h) / _VAR1

        for j, lj in enumerate((l0, l1, l2, l3)):
            d = loc_ref[0][j:j + 1, c0:c0 + cw] - lj
            ad = jnp.abs(d)
            sl1 = jnp.where(ad < 1.0, 0.5 * d * d, ad - 0.5)
            lossl_acc += _padw(jnp.where(pos, sl1, 0.0), cw)

    lossl_b = jnp.sum(lossl_acc)
    npos_b = jnp.sum(npos_acc)
    lane = jax.lax.broadcasted_iota(jnp.int32, (1, 128), 1)
    stats_ref[0, 0:1, :] = jnp.where(
        lane == 0, lossl_b, jnp.where(lane == 2, npos_b, 0.0))


def _ce_mine_body(conf_ref, ct_ref, stats_ref, out_ref, lcb_scr, acc_scr):
    b = pl.program_id(0)
    nb = pl.num_programs(0)
    C = conf_ref.shape[1]
    P = conf_ref.shape[2]
    lane = jax.lax.broadcasted_iota(jnp.int32, (1, 128), 1)

    lcp_acc = jnp.zeros((1, _W), jnp.float32)
    vals = []
    for c0, cw in _chunks(P):
        c = conf_ref[0][:, c0:c0 + cw]
        cti = ct_ref[0][0:1, c0:c0 + cw]
        pos = cti > 0
        m = jnp.max(c, axis=0, keepdims=True)
        e = jnp.exp(c - m)
        s = jax.lax.dot_general(
            jnp.ones((1, C), jnp.float32), e, (((1,), (0,)), ((), ())),
            preferred_element_type=jnp.float32)
        lse = m + jnp.log(s)
        cls_iota = jax.lax.broadcasted_iota(jnp.int32, (C, cw), 0)
        ltgt = jnp.sum(jnp.where(cls_iota == cti, c, 0.0),
                       axis=0, keepdims=True)
        ce = lse - ltgt
        lcp_acc += _padw(jnp.where(pos, ce, 0.0), cw)
        cwp = ((cw + 127) // 128) * 128
        val = jnp.maximum(ce, 0.0)
        vals.append(jnp.pad(val, ((0, 0), (0, cwp - cw)))
                    if cwp != cw else val)

    lcb_scr[pl.ds(b, 1), :] = jnp.concatenate(vals, axis=1)

    @pl.when(b == 0)
    def _():
        acc_scr[0:1, :] = jnp.zeros((1, 128), jnp.float32)

    acc_scr[0:1, :] += jnp.where(lane == 1, jnp.sum(lcp_acc), 0.0)

    @pl.when(b == nb - 1)
    def _():
        stats = stats_ref[:, :]
        tot = jnp.sum(stats, axis=0, keepdims=True) + acc_scr[0:1, :]
        ll = jnp.sum(jnp.where(lane == 0, tot, 0.0))
        lcp = jnp.sum(jnp.where(lane == 1, tot, 0.0))
        n_tot = jnp.sum(jnp.where(lane == 2, tot, 0.0))

        kf = jnp.minimum(jnp.float32(_NEGPOS_RATIO) * stats[:, 2:3],
                         jnp.float32(P - 1))
        v = lcb_scr[:, :]

        lo0 = jnp.zeros((nb, 1), jnp.int32)
        hi0 = jnp.full((nb, 1), 0x7F800000, jnp.int32)

        def step(_, carry):
            lo, hi = carry
            mid = lo + ((hi - lo + 1) >> 1)
            t = jax.lax.bitcast_convert_type(mid, jnp.float32)
            cnt = jnp.sum(jnp.where(v >= t, 1.0, 0.0), axis=1, keepdims=True)
            ok = cnt >= kf
            return jnp.where(ok, mid, lo), jnp.where(ok, hi, mid - 1)

        lo, _hi = jax.lax.fori_loop(0, 31, step, (lo0, hi0))
        tk = jax.lax.bitcast_convert_type(lo, jnp.float32)
        gt = v > tk
        sum_gt = jnp.sum(jnp.where(gt, v, 0.0), axis=1, keepdims=True)
        cnt_gt = jnp.sum(jnp.where(gt, 1.0, 0.0), axis=1, keepdims=True)
        lcn = jnp.sum(sum_gt + (kf - cnt_gt) * tk)

        out_ref[0:1, :] = jnp.where(
            lane == 0, ll / n_tot,
            jnp.where(lane == 1, (lcp + lcn) / n_tot, 0.0))


def _mbox_loss(loc_data, conf_data, priors, targets, interpret=False):
    bs, P, C = conf_data.shape
    T = targets.shape[1]
    Ppad = ((P + 127) // 128) * 128
    loc_t = jnp.swapaxes(loc_data, 1, 2)
    conf_t = jnp.swapaxes(conf_data, 1, 2)
    pri_t = priors[:P, :].T
    tgt_p = jnp.pad(targets, ((0, 0), (0, 0), (0, 128 - targets.shape[2])))

    ct, stats = pl.pallas_call(
        _match_body,
        grid=(bs,),
        in_specs=[
            pl.BlockSpec((1, T, 128), lambda b: (b, 0, 0)),
            pl.BlockSpec((1, 4, P), lambda b: (b, 0, 0)),
            pl.BlockSpec((4, P), lambda b: (0, 0)),
        ],
        out_specs=[
            pl.BlockSpec((1, 1, Ppad), lambda b: (b, 0, 0)),
            pl.BlockSpec((1, 1, 128), lambda b: (b, 0, 0)),
        ],
        out_shape=[
            jax.ShapeDtypeStruct((bs, 1, Ppad), jnp.int32),
            jax.ShapeDtypeStruct((bs, 1, 128), jnp.float32),
        ],
        scratch_shapes=[pltpu.VMEM((T, P), jnp.float32)],
        compiler_params=pltpu.CompilerParams(
            dimension_semantics=("arbitrary",)),
        interpret=interpret,
    )(tgt_p, loc_t, pri_t)

    out = pl.pallas_call(
        _ce_mine_body,
        grid=(bs,),
        in_specs=[
            pl.BlockSpec((1, C, P), lambda b: (b, 0, 0)),
            pl.BlockSpec((1, 1, Ppad), lambda b: (b, 0, 0)),
            pl.BlockSpec((bs, 128), lambda b: (0, 0)),
        ],
        out_specs=pl.BlockSpec((1, 128), lambda b: (0, 0)),
        out_shape=jax.ShapeDtypeStruct((1, 128), jnp.float32),
        scratch_shapes=[
            pltpu.VMEM((bs, Ppad), jnp.float32),
            pltpu.VMEM((1, 128), jnp.float32),
        ],
        compiler_params=pltpu.CompilerParams(
            dimension_semantics=("arbitrary",)),
        interpret=interpret,
    )(conf_t, ct, stats.reshape(bs, 128))
    return out[0, 0], out[0, 1]


def kernel(loc_data, conf_data, priors, targets):
    return _mbox_loss(loc_data, conf_data, priors, targets)
